# baseline (speedup 1.0000x reference)
import functools

import jax
import jax.numpy as jnp
from jax import lax
from jax.experimental import pallas as pl
from jax.experimental.pallas import tpu as pltpu
from jax.experimental.pallas import tpu_sc as plsc

N = 10000
E = 320000
D = 40
DP = 48
NC, NS = 2, 16
NW = NC * NS
EW = E // NW
CH = 128
NCHUNK = EW // CH
TAIL = EW - NCHUNK * CH
RPT = N // NS
ZR = 125

_mesh = plsc.VectorSubcoreMesh(core_axis_name="c", subcore_axis_name="s")
_sc_params = pltpu.CompilerParams(
    use_tc_tiling_on_sc=False, needs_layout_passes=False
)


NH = 10240
HS = NH // NS


@functools.partial(
    pl.kernel,
    out_type=jax.ShapeDtypeStruct((NC, NH), jnp.float32),
    mesh=_mesh,
    scratch_types=[
        pltpu.VMEM((EW,), jnp.int32),
        pltpu.VMEM((NH,), jnp.float32),
        pltpu.VMEM((HS,), jnp.float32),
        pltpu.VMEM((HS,), jnp.float32),
        pltpu.VMEM_SHARED((NS, NH), jnp.float32),
    ],
    compiler_params=_sc_params,
)
def _deg_kernel(dst_hbm, out_hbm, dstbuf, hist, tbuf, abuf, shist):
    cid = lax.axis_index("c")
    sid = lax.axis_index("s")
    wid = sid * NC + cid
    zeros16 = jnp.zeros((16,), jnp.float32)
    ones16 = jnp.ones((16,), jnp.float32)

    def zbody(i, carry):
        hist[pl.ds(pl.multiple_of(i * 16, 16), 16)] = zeros16
        return carry

    lax.fori_loop(0, NH // 16, zbody, 0)
    pltpu.sync_copy(dst_hbm.at[pl.ds(pl.multiple_of(wid * EW, 16), EW)], dstbuf)

    def cbody(i, carry):
        d = dstbuf[pl.ds(pl.multiple_of(i * 16, 16), 16)]
        plsc.addupdate_scatter(hist, [d], ones16)
        return carry

    lax.fori_loop(0, EW // 16, cbody, 0)

    pltpu.sync_copy(hist, shist.at[sid])
    plsc.subcore_barrier()

    def zab(i, carry):
        abuf[pl.ds(pl.multiple_of(i * 16, 16), 16)] = zeros16
        return carry

    lax.fori_loop(0, HS // 16, zab, 0)

    def merge(t, carry):
        pltpu.sync_copy(shist.at[t, pl.ds(sid * HS, HS)], tbuf)

        def addv(i, c2):
            sl = pl.ds(pl.multiple_of(i * 16, 16), 16)
            abuf[sl] = abuf[sl] + tbuf[sl]
            return c2

        lax.fori_loop(0, HS // 16, addv, 0)
        return carry

    lax.fori_loop(0, NS, merge, 0)
    pltpu.sync_copy(abuf, out_hbm.at[cid, pl.ds(sid * HS, HS)])


NBUF = 5
LG = 2
NSTEP = NCHUNK + (NBUF - LG)
NOUTER = -(-NSTEP // NBUF)


@functools.partial(
    pl.kernel,
    out_type=jax.ShapeDtypeStruct((NC, N, DP), jnp.float32),
    mesh=_mesh,
    scratch_types=[
        pltpu.VMEM((EW,), jnp.int32),
        pltpu.VMEM((EW,), jnp.int32),
        pltpu.VMEM((NBUF, CH, DP), jnp.float32),
        pltpu.VMEM((TAIL, DP), jnp.float32),
        pltpu.VMEM((ZR, DP), jnp.float32),
        pltpu.VMEM_SHARED((N, DP), jnp.float32),
        pltpu.SemaphoreType.DMA((NBUF,)),
        pltpu.SemaphoreType.DMA((NBUF,)),
    ],
    compiler_params=_sc_params,
)
def _prop_kernel(z_hbm, src_hbm, dst_hbm, out_hbm, srcb, dstb, rows, trows,
                 zblk, acc, gsem, ssem):
    cid = lax.axis_index("c")
    sid = lax.axis_index("s")
    wid = sid * NC + cid
    zeros16 = jnp.zeros((16,), jnp.float32)

    pltpu.sync_copy(src_hbm.at[pl.ds(pl.multiple_of(wid * EW, 16), EW)], srcb)
    pltpu.sync_copy(dst_hbm.at[pl.ds(pl.multiple_of(wid * EW, 16), EW)], dstb)

    def zb(i, carry):
        zblk[i // (DP // 16), pl.ds((i % (DP // 16)) * 16, 16)] = zeros16
        return carry

    lax.fori_loop(0, ZR * (DP // 16), zb, 0)

    def zc(k, carry):
        pltpu.sync_copy(zblk, acc.at[pl.ds(sid * RPT + k * ZR, ZR)])
        return carry

    lax.fori_loop(0, RPT // ZR, zc, 0)

    def _idx(buf, i, n=CH):
        return buf.at[pl.ds(pl.multiple_of(i * CH, 16), n)]

    for b in range(LG):
        pltpu.async_copy(z_hbm.at[_idx(srcb, b)], rows.at[b], gsem.at[b])
    plsc.subcore_barrier()

    def outer(j, carry):
        for k in range(NBUF):
            i = j * NBUF + k
            bg = (k + LG) % NBUF

            @pl.when(
                jnp.logical_and(i >= NBUF - LG, i <= NCHUNK - 1 + (NBUF - LG))
            )
            def _retire_scatter():
                pltpu.make_async_copy(
                    rows.at[bg], acc.at[_idx(dstb, i - (NBUF - LG))], ssem.at[bg]
                ).wait()

            @pl.when(i <= NCHUNK - 1 - LG)
            def _launch_gather():
                pltpu.async_copy(
                    z_hbm.at[_idx(srcb, i + LG)], rows.at[bg], gsem.at[bg]
                )

            @pl.when(i <= NCHUNK - 1)
            def _gather_to_scatter():
                pltpu.make_async_copy(
                    z_hbm.at[_idx(srcb, i)], rows.at[k], gsem.at[k]
                ).wait()
                pltpu.async_copy(
                    rows.at[k], acc.at[_idx(dstb, i)], ssem.at[k], add=True
                )

        return carry

    lax.fori_loop(0, NOUTER, outer, 0)

    pltpu.async_copy(
        z_hbm.at[_idx(srcb, NCHUNK, TAIL)], trows, gsem.at[0]
    ).wait()
    pltpu.sync_copy(trows, acc.at[_idx(dstb, NCHUNK, TAIL)], add=True)

    plsc.subcore_barrier()
    pltpu.sync_copy(
        acc.at[pl.ds(sid * RPT, RPT)],
        out_hbm.at[cid, pl.ds(sid * RPT, RPT)],
    )


GRID = 10
BN = N // GRID


def _dinv_body(deg_ref, dinv_ref):
    deg = deg_ref[0] + deg_ref[1] + 1.0
    dinv_ref[...] = lax.rsqrt(deg)[:, None]


_dinv = pl.pallas_call(
    _dinv_body,
    out_shape=jax.ShapeDtypeStruct((NH, 1), jnp.float32),
)


def _prep_body(x_ref, w1_ref, w2_ref, dinv_ref, z_ref):
    wt = jnp.dot(w1_ref[...], w2_ref[...], preferred_element_type=jnp.float32)
    y = jnp.dot(x_ref[...], wt, preferred_element_type=jnp.float32)
    z = y * dinv_ref[...]
    z_ref[...] = jnp.concatenate([z, jnp.zeros((BN, DP - D), jnp.float32)], axis=1)


_prep = pl.pallas_call(
    _prep_body,
    grid=(GRID,),
    in_specs=[
        pl.BlockSpec((BN, 128), lambda i: (i, 0)),
        pl.BlockSpec((128, 128), lambda i: (0, 0)),
        pl.BlockSpec((128, D), lambda i: (0, 0)),
        pl.BlockSpec((BN, 1), lambda i: (i, 0)),
    ],
    out_specs=pl.BlockSpec((BN, DP), lambda i: (i, 0)),
    out_shape=jax.ShapeDtypeStruct((N, DP), jnp.float32),
)


def _mid_body(accs_ref, z1_ref, dinv_ref, w2_ref, b1_ref, z2_ref):
    out1 = dinv_ref[...] * (accs_ref[0] + accs_ref[1] + z1_ref[...])
    c = jnp.dot(b1_ref[...], w2_ref[...], preferred_element_type=jnp.float32)
    c48 = jnp.concatenate([c, jnp.zeros((1, DP - D), jnp.float32)], axis=1)
    z2_ref[...] = dinv_ref[...] * (out1 + c48)


_mid = pl.pallas_call(
    _mid_body,
    grid=(GRID,),
    in_specs=[
        pl.BlockSpec((NC, BN, DP), lambda i: (0, i, 0)),
        pl.BlockSpec((BN, DP), lambda i: (i, 0)),
        pl.BlockSpec((BN, 1), lambda i: (i, 0)),
        pl.BlockSpec((128, D), lambda i: (0, 0)),
        pl.BlockSpec((1, 128), lambda i: (0, 0)),
    ],
    out_specs=pl.BlockSpec((BN, DP), lambda i: (i, 0)),
    out_shape=jax.ShapeDtypeStruct((N, DP), jnp.float32),
)


def _final_body(accs_ref, z2_ref, dinv_ref, b2_ref, out_ref):
    logits = (
        dinv_ref[...] * (accs_ref[0] + accs_ref[1] + z2_ref[...]) + b2_ref[...]
    )
    col = lax.broadcasted_iota(jnp.int32, (BN, DP), 1)
    mask = col < D
    lm = jnp.where(mask, logits, jnp.float32(-1e30))
    m = jnp.max(lm, axis=1, keepdims=True)
    e = jnp.where(mask, jnp.exp(lm - m), 0.0)
    s = jnp.sum(e, axis=1, keepdims=True)
    out = lm - m - jnp.log(s)
    out_ref[...] = lax.slice(out, (0, 0), (BN, D))


_final = pl.pallas_call(
    _final_body,
    grid=(GRID,),
    in_specs=[
        pl.BlockSpec((NC, BN, DP), lambda i: (0, i, 0)),
        pl.BlockSpec((BN, DP), lambda i: (i, 0)),
        pl.BlockSpec((BN, 1), lambda i: (i, 0)),
        pl.BlockSpec((1, DP), lambda i: (0, 0)),
    ],
    out_specs=pl.BlockSpec((BN, D), lambda i: (i, 0)),
    out_shape=jax.ShapeDtypeStruct((N, D), jnp.float32),
)


def kernel(x, edge_index, W1, b1, W2, b2):
    src = edge_index[0].astype(jnp.int32)
    dst = edge_index[1].astype(jnp.int32)
    b1r = b1.astype(jnp.float32).reshape(1, -1)
    b2p = jnp.pad(b2.astype(jnp.float32), (0, DP - D)).reshape(1, DP)

    degs = _deg_kernel(dst)
    dinv = _dinv(degs)
    z1 = _prep(x, W1, W2, dinv)
    accs1 = _prop_kernel(z1, src, dst)
    z2 = _mid(accs1, z1, dinv, W2, b1r)
    accs2 = _prop_kernel(z2, src, dst)
    return _final(accs2, z2, dinv, b2p)


# speedup vs baseline: 1.0666x; 1.0666x over previous
import functools

import jax
import jax.numpy as jnp
from jax import lax
from jax.experimental import pallas as pl
from jax.experimental.pallas import tpu as pltpu
from jax.experimental.pallas import tpu_sc as plsc

N = 10000
E = 320000
NPAD = 10240
D = 40
DP = 48
NC, NS = 2, 16
NW = NC * NS
EW = E // NW
CH = 128
NCHUNK = EW // CH
TAIL = EW - NCHUNK * CH
RPT = NPAD // NS

_mesh = plsc.VectorSubcoreMesh(core_axis_name="c", subcore_axis_name="s")
_sc_params = pltpu.CompilerParams(
    use_tc_tiling_on_sc=False, needs_layout_passes=False
)


@functools.partial(
    pl.kernel,
    out_type=jax.ShapeDtypeStruct((NW, NPAD), jnp.float32),
    mesh=_mesh,
    scratch_types=[
        pltpu.VMEM((EW,), jnp.int32),
        pltpu.VMEM((NPAD,), jnp.float32),
    ],
    compiler_params=_sc_params,
)
def _deg_kernel(dst_hbm, out_hbm, dstbuf, hist):
    cid = lax.axis_index("c")
    sid = lax.axis_index("s")
    wid = sid * NC + cid
    zeros16 = jnp.zeros((16,), jnp.float32)
    ones16 = jnp.ones((16,), jnp.float32)

    def zbody(i, carry):
        hist[pl.ds(pl.multiple_of(i * 16, 16), 16)] = zeros16
        return carry

    lax.fori_loop(0, NPAD // 16, zbody, 0)
    pltpu.sync_copy(dst_hbm.at[pl.ds(pl.multiple_of(wid * EW, 16), EW)], dstbuf)

    def cbody(i, carry):
        d = dstbuf[pl.ds(pl.multiple_of(i * 16, 16), 16)]
        plsc.addupdate_scatter(hist, [d], ones16)
        return carry

    lax.fori_loop(0, EW // 16, cbody, 0)
    pltpu.sync_copy(hist, out_hbm.at[wid])


NBUF = 5
LG = 2
NSTEP = NCHUNK + (NBUF - LG)
NOUTER = -(-NSTEP // NBUF)


@functools.partial(
    pl.kernel,
    out_type=jax.ShapeDtypeStruct((NC, NPAD, DP), jnp.float32),
    mesh=_mesh,
    scratch_types=[
        pltpu.VMEM((EW,), jnp.int32),
        pltpu.VMEM((EW,), jnp.int32),
        pltpu.VMEM((NBUF, CH, DP), jnp.float32),
        pltpu.VMEM((TAIL, DP), jnp.float32),
        pltpu.VMEM((128, DP), jnp.float32),
        pltpu.VMEM_SHARED((NPAD, DP), jnp.float32),
        pltpu.SemaphoreType.DMA((NBUF,)),
        pltpu.SemaphoreType.DMA((NBUF,)),
    ],
    compiler_params=_sc_params,
)
def _prop_kernel(z_hbm, src_hbm, dst_hbm, out_hbm, srcb, dstb, rows, trows,
                 zblk, acc, gsem, ssem):
    cid = lax.axis_index("c")
    sid = lax.axis_index("s")
    wid = sid * NC + cid
    zeros16 = jnp.zeros((16,), jnp.float32)

    pltpu.sync_copy(src_hbm.at[pl.ds(pl.multiple_of(wid * EW, 16), EW)], srcb)
    pltpu.sync_copy(dst_hbm.at[pl.ds(pl.multiple_of(wid * EW, 16), EW)], dstb)

    def zb(i, carry):
        zblk[i // (DP // 16), pl.ds((i % (DP // 16)) * 16, 16)] = zeros16
        return carry

    lax.fori_loop(0, 128 * (DP // 16), zb, 0)

    def zc(k, carry):
        pltpu.sync_copy(zblk, acc.at[pl.ds(sid * RPT + k * 128, 128)])
        return carry

    lax.fori_loop(0, RPT // 128, zc, 0)

    def _idx(buf, i, n=CH):
        return buf.at[pl.ds(pl.multiple_of(i * CH, 16), n)]

    for b in range(LG):
        pltpu.async_copy(z_hbm.at[_idx(srcb, b)], rows.at[b], gsem.at[b])
    plsc.subcore_barrier()

    def outer(j, carry):
        for k in range(NBUF):
            i = j * NBUF + k
            bg = (k + LG) % NBUF

            @pl.when(
                jnp.logical_and(i >= NBUF - LG, i <= NCHUNK - 1 + (NBUF - LG))
            )
            def _retire_scatter():
                pltpu.make_async_copy(
                    rows.at[bg], acc.at[_idx(dstb, i - (NBUF - LG))], ssem.at[bg]
                ).wait()

            @pl.when(i <= NCHUNK - 1 - LG)
            def _launch_gather():
                pltpu.async_copy(
                    z_hbm.at[_idx(srcb, i + LG)], rows.at[bg], gsem.at[bg]
                )

            @pl.when(i <= NCHUNK - 1)
            def _gather_to_scatter():
                pltpu.make_async_copy(
                    z_hbm.at[_idx(srcb, i)], rows.at[k], gsem.at[k]
                ).wait()
                pltpu.async_copy(
                    rows.at[k], acc.at[_idx(dstb, i)], ssem.at[k], add=True
                )

        return carry

    lax.fori_loop(0, NOUTER, outer, 0)

    pltpu.async_copy(
        z_hbm.at[_idx(srcb, NCHUNK, TAIL)], trows, gsem.at[0]
    ).wait()
    pltpu.sync_copy(trows, acc.at[_idx(dstb, NCHUNK, TAIL)], add=True)

    plsc.subcore_barrier()
    pltpu.sync_copy(
        acc.at[pl.ds(sid * RPT, RPT)],
        out_hbm.at[cid, pl.ds(sid * RPT, RPT)],
    )


def _prep_body(x_ref, w1_ref, w2_ref, deg_ref, z_ref, dinv_ref):
    wt = jnp.dot(w1_ref[...], w2_ref[...], preferred_element_type=jnp.float32)
    y = jnp.dot(x_ref[...], wt, preferred_element_type=jnp.float32)
    deg = jnp.sum(deg_ref[...], axis=0) + 1.0
    dinv = lax.rsqrt(deg)
    z = y * dinv[:, None]
    z_ref[...] = jnp.concatenate([z, jnp.zeros((NPAD, DP - D), jnp.float32)], axis=1)
    dinv_ref[...] = dinv[:, None]


_prep = pl.pallas_call(
    _prep_body,
    out_shape=(
        jax.ShapeDtypeStruct((NPAD, DP), jnp.float32),
        jax.ShapeDtypeStruct((NPAD, 1), jnp.float32),
    ),
)


def _mid_body(accs_ref, z1_ref, dinv_ref, w2_ref, b1_ref, z2_ref):
    out1 = dinv_ref[...] * (accs_ref[0] + accs_ref[1] + z1_ref[...])
    c = jnp.dot(b1_ref[...], w2_ref[...], preferred_element_type=jnp.float32)
    c48 = jnp.concatenate([c, jnp.zeros((1, DP - D), jnp.float32)], axis=1)
    z2_ref[...] = dinv_ref[...] * (out1 + c48)


_mid = pl.pallas_call(
    _mid_body,
    out_shape=jax.ShapeDtypeStruct((NPAD, DP), jnp.float32),
)


def _final_body(accs_ref, z2_ref, dinv_ref, b2_ref, out_ref):
    logits = (
        dinv_ref[...] * (accs_ref[0] + accs_ref[1] + z2_ref[...]) + b2_ref[...]
    )
    col = lax.broadcasted_iota(jnp.int32, (NPAD, DP), 1)
    mask = col < D
    lm = jnp.where(mask, logits, jnp.float32(-1e30))
    m = jnp.max(lm, axis=1, keepdims=True)
    e = jnp.where(mask, jnp.exp(lm - m), 0.0)
    s = jnp.sum(e, axis=1, keepdims=True)
    out_ref[...] = lm - m - jnp.log(s)


_final = pl.pallas_call(
    _final_body,
    out_shape=jax.ShapeDtypeStruct((NPAD, DP), jnp.float32),
)


def kernel(x, edge_index, W1, b1, W2, b2):
    src = edge_index[0].astype(jnp.int32)
    dst = edge_index[1].astype(jnp.int32)
    xp = jnp.pad(x.astype(jnp.float32), ((0, NPAD - N), (0, 0)))
    b1r = b1.astype(jnp.float32).reshape(1, -1)
    b2p = jnp.pad(b2.astype(jnp.float32), (0, DP - D)).reshape(1, DP)

    degs = _deg_kernel(dst)
    z1, dinv = _prep(xp, W1, W2, degs)
    accs1 = _prop_kernel(z1, src, dst)
    z2 = _mid(accs1, z1, dinv, W2, b1r)
    accs2 = _prop_kernel(z2, src, dst)
    out48 = _final(accs2, z2, dinv, b2p)
    return out48[:N, :D]


# speedup vs baseline: 1.1320x; 1.0613x over previous
import functools

import jax
import jax.numpy as jnp
from jax import lax
from jax.experimental import pallas as pl
from jax.experimental.pallas import tpu as pltpu
from jax.experimental.pallas import tpu_sc as plsc

N = 10000
E = 320000
NPAD = 10240
D = 40
DP = 48
NC, NS = 2, 16
NW = NC * NS
EW = E // NW
CH = 128
NCHUNK = EW // CH
TAIL = EW - NCHUNK * CH
RPT = NPAD // NS

_mesh = plsc.VectorSubcoreMesh(core_axis_name="c", subcore_axis_name="s")
_sc_params = pltpu.CompilerParams(
    use_tc_tiling_on_sc=False, needs_layout_passes=False
)


@functools.partial(
    pl.kernel,
    out_type=jax.ShapeDtypeStruct((NW, NPAD), jnp.float32),
    mesh=_mesh,
    scratch_types=[
        pltpu.VMEM((EW,), jnp.int32),
        pltpu.VMEM((NPAD,), jnp.float32),
    ],
    compiler_params=_sc_params,
)
def _deg_kernel(edge_hbm, out_hbm, dstbuf, hist):
    cid = lax.axis_index("c")
    sid = lax.axis_index("s")
    wid = sid * NC + cid
    zeros16 = jnp.zeros((16,), jnp.float32)
    ones16 = jnp.ones((16,), jnp.float32)

    def zbody(i, carry):
        hist[pl.ds(pl.multiple_of(i * 16, 16), 16)] = zeros16
        return carry

    lax.fori_loop(0, NPAD // 16, zbody, 0)
    pltpu.sync_copy(
        edge_hbm.at[1, pl.ds(pl.multiple_of(wid * EW, 16), EW)], dstbuf
    )

    def cbody(i, carry):
        d = dstbuf[pl.ds(pl.multiple_of(i * 16, 16), 16)]
        plsc.addupdate_scatter(hist, [d], ones16)
        return carry

    lax.fori_loop(0, EW // 16, cbody, 0)
    pltpu.sync_copy(hist, out_hbm.at[wid])


NBUF = 5
LG = 2
NSTEP = NCHUNK + (NBUF - LG)
NOUTER = -(-NSTEP // NBUF)


@functools.partial(
    pl.kernel,
    out_type=jax.ShapeDtypeStruct((NC, NPAD, DP), jnp.float32),
    mesh=_mesh,
    scratch_types=[
        pltpu.VMEM((EW,), jnp.int32),
        pltpu.VMEM((EW,), jnp.int32),
        pltpu.VMEM((NBUF, CH, DP), jnp.float32),
        pltpu.VMEM((TAIL, DP), jnp.float32),
        pltpu.VMEM((128, DP), jnp.float32),
        pltpu.VMEM_SHARED((NPAD, DP), jnp.float32),
        pltpu.SemaphoreType.DMA((NBUF,)),
        pltpu.SemaphoreType.DMA((NBUF,)),
    ],
    compiler_params=_sc_params,
)
def _prop_kernel(z_hbm, edge_hbm, out_hbm, srcb, dstb, rows, trows,
                 zblk, acc, gsem, ssem):
    cid = lax.axis_index("c")
    sid = lax.axis_index("s")
    wid = sid * NC + cid
    zeros16 = jnp.zeros((16,), jnp.float32)

    pltpu.sync_copy(
        edge_hbm.at[0, pl.ds(pl.multiple_of(wid * EW, 16), EW)], srcb
    )
    pltpu.sync_copy(
        edge_hbm.at[1, pl.ds(pl.multiple_of(wid * EW, 16), EW)], dstb
    )

    def zb(i, carry):
        zblk[i // (DP // 16), pl.ds((i % (DP // 16)) * 16, 16)] = zeros16
        return carry

    lax.fori_loop(0, 128 * (DP // 16), zb, 0)

    def zc(k, carry):
        pltpu.sync_copy(zblk, acc.at[pl.ds(sid * RPT + k * 128, 128)])
        return carry

    lax.fori_loop(0, RPT // 128, zc, 0)

    def _idx(buf, i, n=CH):
        return buf.at[pl.ds(pl.multiple_of(i * CH, 16), n)]

    for b in range(LG):
        pltpu.async_copy(z_hbm.at[_idx(srcb, b)], rows.at[b], gsem.at[b])
    plsc.subcore_barrier()

    def outer(j, carry):
        for k in range(NBUF):
            i = j * NBUF + k
            bg = (k + LG) % NBUF

            @pl.when(
                jnp.logical_and(i >= NBUF - LG, i <= NCHUNK - 1 + (NBUF - LG))
            )
            def _retire_scatter():
                pltpu.make_async_copy(
                    rows.at[bg], acc.at[_idx(dstb, i - (NBUF - LG))], ssem.at[bg]
                ).wait()

            @pl.when(i <= NCHUNK - 1 - LG)
            def _launch_gather():
                pltpu.async_copy(
                    z_hbm.at[_idx(srcb, i + LG)], rows.at[bg], gsem.at[bg]
                )

            @pl.when(i <= NCHUNK - 1)
            def _gather_to_scatter():
                pltpu.make_async_copy(
                    z_hbm.at[_idx(srcb, i)], rows.at[k], gsem.at[k]
                ).wait()
                pltpu.async_copy(
                    rows.at[k], acc.at[_idx(dstb, i)], ssem.at[k], add=True
                )

        return carry

    lax.fori_loop(0, NOUTER, outer, 0)

    pltpu.async_copy(
        z_hbm.at[_idx(srcb, NCHUNK, TAIL)], trows, gsem.at[0]
    ).wait()
    pltpu.sync_copy(trows, acc.at[_idx(dstb, NCHUNK, TAIL)], add=True)

    plsc.subcore_barrier()
    pltpu.sync_copy(
        acc.at[pl.ds(sid * RPT, RPT)],
        out_hbm.at[cid, pl.ds(sid * RPT, RPT)],
    )


def _prep_body(x_ref, w1_ref, w2_ref, deg_ref, z_ref, dinv_ref):
    wt = jnp.dot(w1_ref[...], w2_ref[...], preferred_element_type=jnp.float32)
    y = jnp.dot(x_ref[...], wt, preferred_element_type=jnp.float32)
    deg = jnp.sum(deg_ref[...], axis=0) + 1.0
    dinv = lax.rsqrt(deg)
    z = y * dinv[:, None]
    z_ref[...] = jnp.concatenate([z, jnp.zeros((NPAD, DP - D), jnp.float32)], axis=1)
    dinv_ref[...] = dinv[:, None]


_prep = pl.pallas_call(
    _prep_body,
    out_shape=(
        jax.ShapeDtypeStruct((NPAD, DP), jnp.float32),
        jax.ShapeDtypeStruct((NPAD, 1), jnp.float32),
    ),
)


def _mid_body(accs_ref, z1_ref, dinv_ref, w2_ref, b1_ref, z2_ref):
    out1 = dinv_ref[...] * (accs_ref[0] + accs_ref[1] + z1_ref[...])
    c = jnp.dot(b1_ref[...], w2_ref[...], preferred_element_type=jnp.float32)
    c48 = jnp.concatenate([c, jnp.zeros((1, DP - D), jnp.float32)], axis=1)
    z2_ref[...] = dinv_ref[...] * (out1 + c48)


_mid = pl.pallas_call(
    _mid_body,
    out_shape=jax.ShapeDtypeStruct((NPAD, DP), jnp.float32),
)


def _final_body(accs_ref, z2_ref, dinv_ref, b2_ref, out_ref):
    logits = (
        dinv_ref[...] * (accs_ref[0] + accs_ref[1] + z2_ref[...]) + b2_ref[...]
    )
    col = lax.broadcasted_iota(jnp.int32, (NPAD, DP), 1)
    mask = col < D
    lm = jnp.where(mask, logits, jnp.float32(-1e30))
    m = jnp.max(lm, axis=1, keepdims=True)
    e = jnp.where(mask, jnp.exp(lm - m), 0.0)
    s = jnp.sum(e, axis=1, keepdims=True)
    out_ref[...] = lm - m - jnp.log(s)


_final = pl.pallas_call(
    _final_body,
    out_shape=jax.ShapeDtypeStruct((NPAD, DP), jnp.float32),
)


def kernel(x, edge_index, W1, b1, W2, b2):
    edges = edge_index.astype(jnp.int32)
    xp = jnp.pad(x.astype(jnp.float32), ((0, NPAD - N), (0, 0)))
    b1r = b1.astype(jnp.float32).reshape(1, -1)
    b2p = jnp.pad(b2.astype(jnp.float32), (0, DP - D)).reshape(1, DP)

    degs = _deg_kernel(edges)
    z1, dinv = _prep(xp, W1, W2, degs)
    accs1 = _prop_kernel(z1, edges)
    z2 = _mid(accs1, z1, dinv, W2, b1r)
    accs2 = _prop_kernel(z2, edges)
    out48 = _final(accs2, z2, dinv, b2p)
    return out48[:N, :D]


# speedup vs baseline: 1.1539x; 1.0194x over previous
import functools

import jax
import jax.numpy as jnp
from jax import lax
from jax.experimental import pallas as pl
from jax.experimental.pallas import tpu as pltpu
from jax.experimental.pallas import tpu_sc as plsc

N = 10000
E = 320000
NPAD = 10240
D = 40
DP = 48
NC, NS = 2, 16
NW = NC * NS
EW = E // NW
CH = 128
NCHUNK = EW // CH
TAIL = EW - NCHUNK * CH
RPT = NPAD // NS

_mesh = plsc.VectorSubcoreMesh(core_axis_name="c", subcore_axis_name="s")
_sc_params = pltpu.CompilerParams(
    use_tc_tiling_on_sc=False, needs_layout_passes=False
)


@functools.partial(
    pl.kernel,
    out_type=jax.ShapeDtypeStruct((NW, NPAD), jnp.float32),
    mesh=_mesh,
    scratch_types=[
        pltpu.VMEM((EW,), jnp.int32),
        pltpu.VMEM((NPAD,), jnp.float32),
    ],
    compiler_params=_sc_params,
)
def _deg_kernel(edge_hbm, out_hbm, dstbuf, hist):
    cid = lax.axis_index("c")
    sid = lax.axis_index("s")
    wid = sid * NC + cid
    zeros16 = jnp.zeros((16,), jnp.float32)
    ones16 = jnp.ones((16,), jnp.float32)

    def zbody(i, carry):
        hist[pl.ds(pl.multiple_of(i * 16, 16), 16)] = zeros16
        return carry

    lax.fori_loop(0, NPAD // 16, zbody, 0)
    pltpu.sync_copy(
        edge_hbm.at[1, pl.ds(pl.multiple_of(wid * EW, 16), EW)], dstbuf
    )

    def cbody(i, carry):
        d = dstbuf[pl.ds(pl.multiple_of(i * 16, 16), 16)]
        plsc.addupdate_scatter(hist, [d], ones16)
        return carry

    lax.fori_loop(0, EW // 16, cbody, 0)
    pltpu.sync_copy(hist, out_hbm.at[wid])


NBUF = 5
LG = 2
NSTEP = NCHUNK + (NBUF - LG)
NOUTER = -(-NSTEP // NBUF)


@functools.partial(
    pl.kernel,
    out_type=jax.ShapeDtypeStruct((NC, NPAD, DP), jnp.float32),
    mesh=_mesh,
    scratch_types=[
        pltpu.VMEM((EW,), jnp.int32),
        pltpu.VMEM((EW,), jnp.int32),
        pltpu.VMEM((NBUF, CH, DP), jnp.float32),
        pltpu.VMEM((TAIL, DP), jnp.float32),
        pltpu.VMEM((128, DP), jnp.float32),
        pltpu.VMEM_SHARED((NPAD, DP), jnp.float32),
        pltpu.SemaphoreType.DMA((NBUF,)),
        pltpu.SemaphoreType.DMA((NBUF,)),
    ],
    compiler_params=_sc_params,
)
def _prop_kernel(z_hbm, edge_hbm, out_hbm, srcb, dstb, rows, trows,
                 zblk, acc, gsem, ssem):
    cid = lax.axis_index("c")
    sid = lax.axis_index("s")
    wid = sid * NC + cid
    zeros16 = jnp.zeros((16,), jnp.float32)

    pltpu.sync_copy(
        edge_hbm.at[0, pl.ds(pl.multiple_of(wid * EW, 16), EW)], srcb
    )
    pltpu.sync_copy(
        edge_hbm.at[1, pl.ds(pl.multiple_of(wid * EW, 16), EW)], dstb
    )

    @pl.when(cid == 0)
    def _seed():
        pltpu.sync_copy(
            z_hbm.at[pl.ds(sid * RPT, RPT)], acc.at[pl.ds(sid * RPT, RPT)]
        )

    @pl.when(cid != 0)
    def _zero():
        def zb(i, carry):
            zblk[i // (DP // 16), pl.ds((i % (DP // 16)) * 16, 16)] = zeros16
            return carry

        lax.fori_loop(0, 128 * (DP // 16), zb, 0)

        def zc(k, carry):
            pltpu.sync_copy(zblk, acc.at[pl.ds(sid * RPT + k * 128, 128)])
            return carry

        lax.fori_loop(0, RPT // 128, zc, 0)

    def _idx(buf, i, n=CH):
        return buf.at[pl.ds(pl.multiple_of(i * CH, 16), n)]

    for b in range(LG):
        pltpu.async_copy(z_hbm.at[_idx(srcb, b)], rows.at[b], gsem.at[b])
    plsc.subcore_barrier()

    def outer(j, carry):
        for k in range(NBUF):
            i = j * NBUF + k
            bg = (k + LG) % NBUF

            @pl.when(
                jnp.logical_and(i >= NBUF - LG, i <= NCHUNK - 1 + (NBUF - LG))
            )
            def _retire_scatter():
                pltpu.make_async_copy(
                    rows.at[bg], acc.at[_idx(dstb, i - (NBUF - LG))], ssem.at[bg]
                ).wait()

            @pl.when(i <= NCHUNK - 1 - LG)
            def _launch_gather():
                pltpu.async_copy(
                    z_hbm.at[_idx(srcb, i + LG)], rows.at[bg], gsem.at[bg]
                )

            @pl.when(i <= NCHUNK - 1)
            def _gather_to_scatter():
                pltpu.make_async_copy(
                    z_hbm.at[_idx(srcb, i)], rows.at[k], gsem.at[k]
                ).wait()
                pltpu.async_copy(
                    rows.at[k], acc.at[_idx(dstb, i)], ssem.at[k], add=True
                )

        return carry

    lax.fori_loop(0, NOUTER, outer, 0)

    pltpu.async_copy(
        z_hbm.at[_idx(srcb, NCHUNK, TAIL)], trows, gsem.at[0]
    ).wait()
    pltpu.sync_copy(trows, acc.at[_idx(dstb, NCHUNK, TAIL)], add=True)

    plsc.subcore_barrier()
    pltpu.sync_copy(
        acc.at[pl.ds(sid * RPT, RPT)],
        out_hbm.at[cid, pl.ds(sid * RPT, RPT)],
    )


def _prep_body(x_ref, w1_ref, w2_ref, deg_ref, z_ref, dinv_ref):
    wt = jnp.dot(w1_ref[...], w2_ref[...], preferred_element_type=jnp.float32)
    y = jnp.dot(x_ref[...], wt, preferred_element_type=jnp.float32)
    deg = jnp.sum(deg_ref[...], axis=0) + 1.0
    dinv = lax.rsqrt(deg)
    z = y * dinv[:, None]
    z_ref[...] = jnp.concatenate([z, jnp.zeros((NPAD, DP - D), jnp.float32)], axis=1)
    dinv_ref[...] = dinv[:, None]


_prep = pl.pallas_call(
    _prep_body,
    out_shape=(
        jax.ShapeDtypeStruct((NPAD, DP), jnp.float32),
        jax.ShapeDtypeStruct((NPAD, 1), jnp.float32),
    ),
)


def _mid_body(accs_ref, dinv_ref, w2_ref, b1_ref, z2_ref):
    out1 = dinv_ref[...] * (accs_ref[0] + accs_ref[1])
    c = jnp.dot(b1_ref[...], w2_ref[...], preferred_element_type=jnp.float32)
    c48 = jnp.concatenate([c, jnp.zeros((1, DP - D), jnp.float32)], axis=1)
    z2_ref[...] = dinv_ref[...] * (out1 + c48)


_mid = pl.pallas_call(
    _mid_body,
    out_shape=jax.ShapeDtypeStruct((NPAD, DP), jnp.float32),
)


def _final_body(accs_ref, dinv_ref, b2_ref, out_ref):
    logits = dinv_ref[...] * (accs_ref[0] + accs_ref[1]) + b2_ref[...]
    col = lax.broadcasted_iota(jnp.int32, (NPAD, DP), 1)
    mask = col < D
    lm = jnp.where(mask, logits, jnp.float32(-1e30))
    m = jnp.max(lm, axis=1, keepdims=True)
    e = jnp.where(mask, jnp.exp(lm - m), 0.0)
    s = jnp.sum(e, axis=1, keepdims=True)
    out_ref[...] = lm - m - jnp.log(s)


_final = pl.pallas_call(
    _final_body,
    out_shape=jax.ShapeDtypeStruct((NPAD, DP), jnp.float32),
)


def kernel(x, edge_index, W1, b1, W2, b2):
    edges = edge_index.astype(jnp.int32)
    xp = jnp.pad(x.astype(jnp.float32), ((0, NPAD - N), (0, 0)))
    b1r = b1.astype(jnp.float32).reshape(1, -1)
    b2p = jnp.pad(b2.astype(jnp.float32), (0, DP - D)).reshape(1, DP)

    degs = _deg_kernel(edges)
    z1, dinv = _prep(xp, W1, W2, degs)
    accs1 = _prop_kernel(z1, edges)
    z2 = _mid(accs1, dinv, W2, b1r)
    accs2 = _prop_kernel(z2, edges)
    out48 = _final(accs2, dinv, b2p)
    return out48[:N, :D]
